# baseline (speedup 1.0000x reference)
import functools

import jax
import jax.numpy as jnp
from jax import lax
from jax.experimental import pallas as pl
from jax.experimental.pallas import tpu as pltpu
from jax.experimental.pallas import tpu_sc as plsc

N = 10000
E = 160000
B = 256

NBG = 10
NBLK = N // NBG
EBLK = 2000
EBG = E // EBLK

GW = 32
EPW = E // GW
GCH = 40
GNCH = EPW // GCH
SPT = E // 16
SCH = 40
SNCH = SPT // SCH
NP = 10240
NPT = NP // 16

_f32 = jnp.float32


def _prep_body(ncols, batch_ref, x_ref, u_ref, wer_ref, wec_ref, weu_ref,
               wn1x_ref, bias_ref, t1_ref, t2_ref, ub_ref):
    e_out, n1_out = ncols
    b = batch_ref[0, 0, :]
    oh = (b[:, None] == lax.broadcasted_iota(jnp.int32, (NBLK, B), 1)
          ).astype(_f32)
    ub = oh @ u_ref[...]
    x = x_ref[...]
    t1 = x @ wer_ref[...] + ub @ weu_ref[...] + bias_ref[0:1, :e_out]
    if e_out < 128:
        t1 = jnp.concatenate([t1, jnp.zeros((NBLK, 128 - e_out), _f32)],
                             axis=1)
    t1_ref[...] = t1
    t2a = x @ wec_ref[...]
    t2b = x @ wn1x_ref[...] + bias_ref[1:2, :n1_out]
    t2_ref[...] = jnp.concatenate([t2a, t2b], axis=1)
    ub_ref[...] = ub


def _prep(batch3, x, u, We_r, We_c, We_u, Wn1_x, bias2, e_out, n1_out):
    ni = x.shape[1]
    gi = u.shape[1]
    full = lambda shp: pl.BlockSpec(shp, lambda i: tuple(0 for _ in shp))
    return pl.pallas_call(
        functools.partial(_prep_body, (e_out, n1_out)),
        grid=(NBG,),
        in_specs=[
            pl.BlockSpec((1, 1, NBLK), lambda i: (i, 0, 0)),
            pl.BlockSpec((NBLK, ni), lambda i: (i, 0)),
            full((B, gi)), full((ni, e_out)), full((ni, e_out)),
            full((gi, e_out)), full((ni, n1_out)), full((8, 128)),
        ],
        out_specs=[
            pl.BlockSpec((NBLK, 128), lambda i: (i, 0)),
            pl.BlockSpec((NBLK, e_out + n1_out), lambda i: (i, 0)),
            pl.BlockSpec((NBLK, gi), lambda i: (i, 0)),
        ],
        out_shape=[
            jax.ShapeDtypeStruct((N, 128), _f32),
            jax.ShapeDtypeStruct((N, e_out + n1_out), _f32),
            jax.ShapeDtypeStruct((N, gi), _f32),
        ],
    )(batch3, x, u, We_r, We_c, We_u, Wn1_x, bias2)


GNB = 5
GOUT = GNCH // GNB


def _sc_gather(r1, c1, T1, T2):
    w1 = T1.shape[1]
    w2 = T2.shape[1]
    mesh = plsc.VectorSubcoreMesh(core_axis_name="c", subcore_axis_name="s")

    @functools.partial(
        pl.kernel, mesh=mesh,
        out_type=[jax.ShapeDtypeStruct((E, w1), _f32),
                  jax.ShapeDtypeStruct((E, w2), _f32)],
        scratch_types=[
            pltpu.VMEM((EPW,), jnp.int32),
            pltpu.VMEM((EPW,), jnp.int32),
            pltpu.VMEM((GNB, GCH, w1), _f32),
            pltpu.VMEM((GNB, GCH, w2), _f32),
        ] + [pltpu.SemaphoreType.DMA] * (4 * GNB),
    )
    def k(r_hbm, c_hbm, t1_hbm, t2_hbm, g1_hbm, g2_hbm,
          ridx, cidx, buf1, buf2, *sem):
        gs1 = sem[0:GNB]
        gs2 = sem[GNB:2 * GNB]
        os1 = sem[2 * GNB:3 * GNB]
        os2 = sem[3 * GNB:4 * GNB]
        cid = lax.axis_index("c")
        sid = lax.axis_index("s")
        wid = sid * 2 + cid
        wbase = wid * EPW
        pltpu.sync_copy(r_hbm.at[pl.ds(wbase, EPW)], ridx)
        pltpu.sync_copy(c_hbm.at[pl.ds(wbase, EPW)], cidx)

        def outer(t, carry):
            ins = []
            for b in range(GNB):
                off = (t * GNB + b) * GCH
                cp1 = pltpu.async_copy(
                    t1_hbm.at[ridx.at[pl.ds(off, GCH)]], buf1.at[b], gs1[b])
                cp2 = pltpu.async_copy(
                    t2_hbm.at[cidx.at[pl.ds(off, GCH)]], buf2.at[b], gs2[b])
                ins.append((cp1, cp2, off))
            outs = []
            for b, (cp1, cp2, off) in enumerate(ins):
                cp1.wait()
                cp2.wait()
                outs.append(pltpu.async_copy(
                    buf1.at[b], g1_hbm.at[pl.ds(wbase + off, GCH)], os1[b]))
                outs.append(pltpu.async_copy(
                    buf2.at[b], g2_hbm.at[pl.ds(wbase + off, GCH)], os2[b]))
            for o in outs:
                o.wait()
            return carry

        lax.fori_loop(0, GOUT, outer, 0)

    return k(r1, c1, T1, T2)


SNB = 5
SOUT = SNCH // SNB


def _sc_scatter(r1, me, zeros_nw):
    w = me.shape[2]
    mesh = plsc.VectorSubcoreMesh(core_axis_name="c", subcore_axis_name="s")

    @functools.partial(
        pl.kernel, mesh=mesh,
        out_type=[jax.ShapeDtypeStruct((NP, w), _f32),
                  jax.ShapeDtypeStruct((NP, w), _f32)],
        scratch_types=[
            pltpu.VMEM((SPT,), jnp.int32),
            pltpu.VMEM((SNB, SCH, w), _f32),
            pltpu.VMEM_SHARED((NP, w), _f32),
        ] + [pltpu.SemaphoreType.DMA] * SNB,
    )
    def k(r_hbm, me_hbm, z_hbm, smsg_hbm, se_hbm, idx_all, dbuf, acc, *sems):
        cid = lax.axis_index("c")
        sid = lax.axis_index("s")
        pltpu.sync_copy(z_hbm.at[pl.ds(sid * NPT, NPT)],
                        acc.at[pl.ds(sid * NPT, NPT)])
        pltpu.sync_copy(r_hbm.at[pl.ds(sid * SPT, SPT)], idx_all)
        plsc.subcore_barrier()

        def outer(t, carry):
            for b in range(SNB):
                base = sid * SPT + (t * SNB + b) * SCH

                @pl.when(cid == 0)
                def _():
                    pltpu.async_copy(me_hbm.at[0, pl.ds(base, SCH)],
                                     dbuf.at[b], sems[b])

                @pl.when(cid == 1)
                def _():
                    pltpu.async_copy(me_hbm.at[1, pl.ds(base, SCH)],
                                     dbuf.at[b], sems[b])

            for b in range(SNB):
                off = (t * SNB + b) * SCH
                base = sid * SPT + off
                pltpu.make_async_copy(me_hbm.at[0, pl.ds(base, SCH)],
                                      dbuf.at[b], sems[b]).wait()
                pltpu.sync_copy(dbuf.at[b],
                                acc.at[idx_all.at[pl.ds(off, SCH)]], add=True)
            return carry

        lax.fori_loop(0, SOUT, outer, 0)
        plsc.subcore_barrier()

        @pl.when(cid == 0)
        def _():
            pltpu.sync_copy(acc.at[pl.ds(sid * NPT, NPT)],
                            smsg_hbm.at[pl.ds(sid * NPT, NPT)])

        @pl.when(cid == 1)
        def _():
            pltpu.sync_copy(acc.at[pl.ds(sid * NPT, NPT)],
                            se_hbm.at[pl.ds(sid * NPT, NPT)])

    return k(r1, me, zeros_nw)


def _edge_body(cfg, st_ref, wee_ref, wn1e_ref, g1_ref, g2_ref, ea_ref,
               me_ref, stats_ref=None):
    e_out, ein, act, stats = cfg
    s = st_ref[0:1, :ein]
    t = st_ref[1:2, :ein]
    ea = ea_ref[0] * s + t
    e1 = g1_ref[:, :e_out] + g2_ref[:, :e_out] + ea @ wee_ref[...]
    if act:
        e1 = jnp.maximum(e1, 0.0)
    msg = g2_ref[:, e_out:] + e1 @ wn1e_ref[...]
    if act:
        msg = jnp.maximum(msg, 0.0)
    if e_out < 128:
        pad = jnp.zeros((msg.shape[0], 128 - e_out), _f32)
        msg = jnp.concatenate([msg, pad], axis=1)
        e1 = jnp.concatenate([e1, pad], axis=1)
    me_ref[0] = msg
    me_ref[1] = e1
    if stats:
        blk = jnp.concatenate(
            [jnp.sum(e1, axis=0)[None, :], jnp.sum(e1 * e1, axis=0)[None, :],
             jnp.zeros((6, e_out), _f32)], axis=0)

        @pl.when(pl.program_id(0) == 0)
        def _():
            stats_ref[...] = blk

        @pl.when(pl.program_id(0) != 0)
        def _():
            stats_ref[...] += blk


def _edge(st, We_e, Wn1_e, G1, G2, ea3, ea_col, act, stats):
    e_out = We_e.shape[1]
    n1_out = G2.shape[1] - e_out
    ein = We_e.shape[0]
    full = lambda shp: pl.BlockSpec(shp, lambda i: tuple(0 for _ in shp))
    out_shape = [jax.ShapeDtypeStruct((2, E, 128), _f32)]
    out_specs = [pl.BlockSpec((2, EBLK, 128), lambda i: (0, i, 0))]
    if stats:
        out_shape.append(jax.ShapeDtypeStruct((8, 128), _f32))
        out_specs.append(pl.BlockSpec((8, 128), lambda i: (0, 0)))
    res = pl.pallas_call(
        functools.partial(_edge_body, (e_out, ein, act, stats)),
        grid=(EBG,),
        in_specs=[
            full((8, 128)), full((ein, e_out)), full((e_out, n1_out)),
            pl.BlockSpec((EBLK, 128), lambda i: (i, 0)),
            pl.BlockSpec((EBLK, e_out + n1_out), lambda i: (i, 0)),
            pl.BlockSpec((1, EBLK, ein), lambda i, _c=ea_col: (_c, i, 0)),
        ],
        out_specs=out_specs,
        out_shape=out_shape,
    )(st, We_e, Wn1_e, G1, G2, ea3)
    if stats:
        return res
    return res[0], None


def _node_body(cfg, batchT_ref, x_ref, sm_ref, se_ref, ub_ref, u_ref,
               wn2x_ref, wn2a_ref, wn2u_ref, wgn_ref, wge_ref, wgu_ref,
               bias_ref, stats_ref, xo_ref, uo_ref, sto_ref=None):
    n2, g_out, e_out, act = cfg
    x_new = (x_ref[...] @ wn2x_ref[...] + sm_ref[0:N, :] @ wn2a_ref[...]
             + ub_ref[...] @ wn2u_ref[...] + bias_ref[0:1, :n2])
    if act:
        x_new = jnp.maximum(x_new, 0.0)
    ohT = (batchT_ref[...] == lax.broadcasted_iota(jnp.int32, (B, N), 0)
           ).astype(_f32)
    nagg = ohT @ x_new
    eagg = ohT @ se_ref[0:N, :]
    u_new = (nagg @ wgn_ref[...] + eagg @ wge_ref[...]
             + u_ref[...] @ wgu_ref[...] + bias_ref[1:2, :g_out])
    if act:
        u_new = jnp.maximum(u_new, 0.0)
    if act:
        mx = jnp.mean(x_new, axis=0, keepdims=True)
        vx = jnp.mean(x_new * x_new, axis=0, keepdims=True) - mx * mx
        x_new = ((x_new - mx) / jnp.sqrt(vx + 1e-5)
                 * bias_ref[2:3, :n2] + bias_ref[3:4, :n2])
        mu = jnp.mean(u_new, axis=0, keepdims=True)
        vu = jnp.mean(u_new * u_new, axis=0, keepdims=True) - mu * mu
        u_new = ((u_new - mu) / jnp.sqrt(vu + 1e-5)
                 * bias_ref[4:5, :g_out] + bias_ref[5:6, :g_out])
        me_ = stats_ref[0:1, :] / E
        ve = stats_ref[1:2, :] / E - me_ * me_
        s_e = bias_ref[6:7, :] / jnp.sqrt(ve + 1e-5)
        t_e = bias_ref[7:8, :] - me_ * s_e
        sto_ref[...] = jnp.concatenate([s_e, t_e, jnp.zeros((6, 128), _f32)],
                                       axis=0)
    xo_ref[...] = x_new
    uo_ref[...] = u_new


def _node(batchT, x, S_msg, S_e, ub, u, Wn2_x, Wn2_a, Wn2_u, Wg_n, Wg_e,
          Wg_u, bias8, stats, act):
    ni = x.shape[1]
    n1 = S_msg.shape[1]
    e_out = S_e.shape[1]
    gi = u.shape[1]
    n2 = Wn2_x.shape[1]
    g_out = Wg_n.shape[1]
    out_shape = [jax.ShapeDtypeStruct((N, n2), _f32),
                 jax.ShapeDtypeStruct((B, g_out), _f32)]
    if act:
        out_shape.append(jax.ShapeDtypeStruct((8, 128), _f32))
    return pl.pallas_call(
        functools.partial(_node_body, (n2, g_out, e_out, act)),
        out_shape=out_shape,
    )(batchT, x, S_msg, S_e, ub, u, Wn2_x, Wn2_a, Wn2_u, Wg_n, Wg_e, Wg_u,
      bias8, stats)


def _pad_rows(vecs, width=128):
    rows = []
    for v in vecs:
        v = jnp.asarray(v, _f32).reshape(-1)
        rows.append(jnp.pad(v, (0, width - v.shape[0]))[None, :])
    rows.append(jnp.zeros((8 - len(vecs), width), _f32))
    return jnp.concatenate(rows, axis=0)


def kernel(x, edge_index, edge_attr, u, batch, params):
    r = edge_index[0]
    c = edge_index[1]
    batch3 = batch.reshape(NBG, 1, NBLK)
    batchT = batch.reshape(1, N)

    st = None
    ea3 = edge_attr[None]
    ea_col = 0
    num_layers = 3
    for i in range(num_layers):
        p = params["layers"][i]
        act = i < num_layers - 1
        ni = x.shape[1]
        ein = ea3.shape[2]
        gi = u.shape[1]
        We, Wn1, Wn2, Wg = p["We"], p["Wn1"], p["Wn2"], p["Wg"]
        e_out = We.shape[1]
        n1_out = Wn1.shape[1]
        We_r, We_c = We[:ni], We[ni:2 * ni]
        We_e, We_u = We[2 * ni:2 * ni + ein], We[2 * ni + ein:]
        Wn1_x, Wn1_e = Wn1[:ni], Wn1[ni:]
        if st is None:
            st = _pad_rows([jnp.ones((ein,), _f32), jnp.zeros((ein,), _f32)])

        bias2 = _pad_rows([p["be"], p["bn1"]])
        T1, T2, ub = _prep(batch3, x, u, We_r, We_c, We_u, Wn1_x, bias2,
                           e_out, n1_out)
        G1, G2 = _sc_gather(r, c, T1, T2)
        me, stats = _edge(st, We_e, Wn1_e, G1, G2, ea3, ea_col, act, act)
        zeros_nw = jnp.zeros((NP, 128), _f32)
        S_msg, S_e = _sc_scatter(r, me, zeros_nw)

        Wn2_x = Wn2[:ni]
        Wn2_a = Wn2[ni:ni + n1_out]
        Wn2_u = Wn2[ni + n1_out:]
        n2 = Wn2.shape[1]
        g_out = Wg.shape[1]
        Wg_n = Wg[:n2]
        Wg_e = Wg[n2:n2 + e_out]
        Wg_u = Wg[n2 + e_out:]
        if n1_out < 128:
            Wn2_a = jnp.pad(Wn2_a, ((0, 128 - n1_out), (0, 0)))
        if e_out < 128:
            Wg_e = jnp.pad(Wg_e, ((0, 128 - e_out), (0, 0)))
        if act:
            bp = params["bns"][i]
            bias8 = _pad_rows([p["bn2"], p["bg"], bp["gx"], bp["bx"],
                               bp["gu"], bp["bu"], bp["ge"], bp["be_"]])
            x, u, st = _node(batchT, x, S_msg, S_e, ub, u, Wn2_x, Wn2_a,
                             Wn2_u, Wg_n, Wg_e, Wg_u, bias8, stats, True)
        else:
            bias8 = _pad_rows([p["bn2"], p["bg"]])
            x, u = _node(batchT, x, S_msg, S_e, ub, u, Wn2_x, Wn2_a, Wn2_u,
                         Wg_n, Wg_e, Wg_u, bias8,
                         jnp.zeros((8, 128), _f32), False)
        ea3 = me
        ea_col = 1

    return (x, ea3[1, :, :e_out], u)
